# Initial kernel scaffold; baseline (speedup 1.0000x reference)
import jax
import jax.numpy as jnp
from jax.experimental import pallas as pl


def kernel(x, edge_index, W1, b1, W2, b2):
    raise NotImplementedError("write your pallas kernel here")


# speedup vs baseline: 35.6059x; 35.6059x over previous
import jax
import jax.numpy as jnp
from jax import lax
from jax.experimental import pallas as pl
from jax.experimental.pallas import tpu as pltpu
from jax.experimental.pallas import tpu_sc as plsc

N_NODES = 100000
N_EDGES = 3200000
LANES = 16

NC, NS = 2, 16
NW = NC * NS
SUB = 8
CHUNK_E = SUB * 128
CHUNKS_PER_W = 98
EPAD = NW * CHUNKS_PER_W * CHUNK_E
ROWS = EPAD // 128
ROWS_PER_W = ROWS // NW
ACC_ROWS = 100096
ZROWS_PER_TILE = ACC_ROWS // NS

_mesh = plsc.VectorSubcoreMesh(
    core_axis_name="c", subcore_axis_name="s", num_cores=NC, num_subcores=NS
)


def _agg_body(table, src_r, dst_r, zeros, out, acc, sidx, didx, rows, sem):
    cid = lax.axis_index("c")
    sid = lax.axis_index("s")
    wid = cid * NS + sid
    z0 = sid * ZROWS_PER_TILE
    pltpu.sync_copy(zeros.at[pl.ds(z0, ZROWS_PER_TILE)],
                    acc.at[pl.ds(z0, ZROWS_PER_TILE)])
    plsc.subcore_barrier()

    rbase = wid * ROWS_PER_W

    def chunk(k, carry):
        r0 = rbase + k * SUB
        pltpu.sync_copy(src_r.at[pl.ds(r0, SUB)], sidx)
        pltpu.sync_copy(dst_r.at[pl.ds(r0, SUB)], didx)
        descs = [
            pltpu.async_copy(table.at[sidx.at[j]],
                             rows.at[pl.ds(j * 128, 128)], sem)
            for j in range(SUB)
        ]
        for d in descs:
            d.wait()
        for j in range(SUB):
            pltpu.sync_copy(rows.at[pl.ds(j * 128, 128)],
                            acc.at[didx.at[j]], add=True)
        return carry

    lax.fori_loop(0, CHUNKS_PER_W, chunk, 0)
    plsc.subcore_barrier()
    pltpu.sync_copy(acc.at[pl.ds(z0, ZROWS_PER_TILE)],
                    out.at[cid, pl.ds(z0, ZROWS_PER_TILE)])


def _deg_body(dst_r, zeros, ones, out, acc, didx, rows, sem):
    cid = lax.axis_index("c")
    sid = lax.axis_index("s")
    wid = cid * NS + sid
    z0 = sid * ZROWS_PER_TILE
    pltpu.sync_copy(zeros.at[pl.ds(z0, ZROWS_PER_TILE)],
                    acc.at[pl.ds(z0, ZROWS_PER_TILE)])
    pltpu.sync_copy(ones, rows)
    plsc.subcore_barrier()

    rbase = wid * ROWS_PER_W

    def chunk(k, carry):
        r0 = rbase + k * SUB
        pltpu.sync_copy(dst_r.at[pl.ds(r0, SUB)], didx)
        for j in range(SUB):
            pltpu.sync_copy(rows, acc.at[didx.at[j]], add=True)
        return carry

    lax.fori_loop(0, CHUNKS_PER_W, chunk, 0)
    plsc.subcore_barrier()
    pltpu.sync_copy(acc.at[pl.ds(z0, ZROWS_PER_TILE)],
                    out.at[cid, pl.ds(z0, ZROWS_PER_TILE)])


_PART = jax.ShapeDtypeStruct((NC, ACC_ROWS, LANES), jnp.float32)

_sc_params = pltpu.CompilerParams(use_tc_tiling_on_sc=False)

_agg = pl.kernel(
    _agg_body,
    out_type=_PART,
    mesh=_mesh,
    compiler_params=_sc_params,
    scratch_types=[
        pltpu.VMEM_SHARED((ACC_ROWS, LANES), jnp.float32),
        pltpu.VMEM((SUB, 128), jnp.int32),
        pltpu.VMEM((SUB, 128), jnp.int32),
        pltpu.VMEM((CHUNK_E, LANES), jnp.float32),
        pltpu.SemaphoreType.DMA,
    ],
)

_deg = pl.kernel(
    _deg_body,
    out_type=_PART,
    mesh=_mesh,
    compiler_params=_sc_params,
    scratch_types=[
        pltpu.VMEM_SHARED((ACC_ROWS, LANES), jnp.float32),
        pltpu.VMEM((SUB, 128), jnp.int32),
        pltpu.VMEM((128, LANES), jnp.float32),
        pltpu.SemaphoreType.DMA,
    ],
)

_BLK = 2000
_GRID = N_NODES // _BLK


def _tc1_body(deg_ref, x_ref, dinv_ref, y1_ref):
    d = deg_ref[0] + deg_ref[1] + 1.0
    dinv = lax.rsqrt(d)
    dinv_ref[...] = dinv
    y1_ref[...] = dinv * x_ref[...]


def _tc2_body(dinv_ref, agg_ref, y1_ref, w1_ref, b1_ref, y2lo_ref, y2hi_ref):
    dinv = dinv_ref[...]
    p1 = dinv * (agg_ref[0] + agg_ref[1] + y1_ref[...])
    h = jnp.dot(p1, w1_ref[...], preferred_element_type=jnp.float32)
    h = jnp.maximum(h + b1_ref[...], 0.0)
    y2lo_ref[...] = dinv * h[:, :LANES]
    y2hi_ref[...] = dinv * h[:, LANES:]


def _tc3_body(dinv_ref, alo_ref, ahi_ref, y2lo_ref, y2hi_ref, w2_ref,
              b2_ref, o_ref):
    dinv = dinv_ref[...]
    plo = dinv * (alo_ref[0] + alo_ref[1] + y2lo_ref[...])
    phi = dinv * (ahi_ref[0] + ahi_ref[1] + y2hi_ref[...])
    p2 = jnp.concatenate([plo, phi], axis=1)
    o = jnp.dot(p2, w2_ref[...], preferred_element_type=jnp.float32)
    o_ref[...] = jnp.maximum(o + b2_ref[...], 0.0)


def _vec_spec():
    return pl.BlockSpec((_BLK, LANES), lambda i: (i, 0))


def _part_spec():
    return pl.BlockSpec((NC, _BLK, LANES), lambda i: (0, i, 0))


def _full_spec(shape):
    return pl.BlockSpec(shape, lambda i: tuple(0 for _ in shape))


_tc1 = pl.pallas_call(
    _tc1_body,
    grid=(_GRID,),
    in_specs=[_part_spec(), _vec_spec()],
    out_specs=[_vec_spec(), _vec_spec()],
    out_shape=[jax.ShapeDtypeStruct((N_NODES, LANES), jnp.float32)] * 2,
)

_tc2 = pl.pallas_call(
    _tc2_body,
    grid=(_GRID,),
    in_specs=[_vec_spec(), _part_spec(), _vec_spec(),
              _full_spec((16, 32)), _full_spec((1, 32))],
    out_specs=[_vec_spec(), _vec_spec()],
    out_shape=[jax.ShapeDtypeStruct((N_NODES, LANES), jnp.float32)] * 2,
)

_tc3 = pl.pallas_call(
    _tc3_body,
    grid=(_GRID,),
    in_specs=[_vec_spec(), _part_spec(), _part_spec(), _vec_spec(),
              _vec_spec(), _full_spec((32, 64)), _full_spec((1, 64))],
    out_specs=pl.BlockSpec((_BLK, 64), lambda i: (i, 0)),
    out_shape=jax.ShapeDtypeStruct((N_NODES, 64), jnp.float32),
)


def kernel(x, edge_index, W1, b1, W2, b2):
    src = edge_index[0]
    dst = edge_index[1]
    pad = EPAD - N_EDGES
    src_p = jnp.concatenate([src, jnp.zeros((pad,), jnp.int32)])
    dst_p = jnp.concatenate([dst, jnp.full((pad,), N_NODES, jnp.int32)])
    src_r = src_p.reshape(ROWS, 128)
    dst_r = dst_p.reshape(ROWS, 128)
    zeros_sh = jnp.zeros((ACC_ROWS, LANES), jnp.float32)
    ones_blk = jnp.ones((128, LANES), jnp.float32)

    deg = _deg(dst_r, zeros_sh, ones_blk)[:, :N_NODES]
    dinv, y1 = _tc1(deg, x)
    agg1 = _agg(y1, src_r, dst_r, zeros_sh)[:, :N_NODES]
    y2lo, y2hi = _tc2(dinv, agg1, y1, W1, b1.reshape(1, 32))
    a2lo = _agg(y2lo, src_r, dst_r, zeros_sh)[:, :N_NODES]
    a2hi = _agg(y2hi, src_r, dst_r, zeros_sh)[:, :N_NODES]
    return _tc3(dinv, a2lo, a2hi, y2lo, y2hi, W2, b2.reshape(1, 64))


# speedup vs baseline: 43.4688x; 1.2208x over previous
import jax
import jax.numpy as jnp
from jax import lax
from jax.experimental import pallas as pl
from jax.experimental.pallas import tpu as pltpu
from jax.experimental.pallas import tpu_sc as plsc

N_NODES = 100000
N_EDGES = 3200000
LANES = 16

NC, NS = 2, 16
NW = NC * NS
SUB = 4
CHUNK_E = SUB * 128
CHUNKS_PER_W = 196
EPAD = NW * CHUNKS_PER_W * CHUNK_E
ROWS = EPAD // 128
ROWS_PER_W = ROWS // NW
ACC_ROWS = 100096
ZROWS_PER_TILE = ACC_ROWS // NS

_mesh = plsc.VectorSubcoreMesh(
    core_axis_name="c", subcore_axis_name="s", num_cores=NC, num_subcores=NS
)


def _agg_body(table, src_r, dst_r, zeros, out, acc,
              sidx_a, didx_a, rows_a, sem_a,
              sidx_b, didx_b, rows_b, sem_b):
    cid = lax.axis_index("c")
    sid = lax.axis_index("s")
    wid = cid * NS + sid
    z0 = sid * ZROWS_PER_TILE
    pltpu.sync_copy(zeros.at[pl.ds(z0, ZROWS_PER_TILE)],
                    acc.at[pl.ds(z0, ZROWS_PER_TILE)])
    plsc.subcore_barrier()

    rbase = wid * ROWS_PER_W

    def load_and_fire(c, sidx, didx, rows, sem):
        r0 = rbase + c * SUB
        pltpu.sync_copy(src_r.at[pl.ds(r0, SUB)], sidx)
        pltpu.sync_copy(dst_r.at[pl.ds(r0, SUB)], didx)
        for j in range(SUB):
            pltpu.async_copy(table.at[sidx.at[j]],
                             rows.at[pl.ds(j * 128, 128)], sem)

    def drain_and_scatter(sidx, didx, rows, sem):
        for j in range(SUB):
            pltpu.make_async_copy(table.at[sidx.at[j]],
                                  rows.at[pl.ds(j * 128, 128)], sem).wait()
        for j in range(SUB):
            pltpu.sync_copy(rows.at[pl.ds(j * 128, 128)],
                            acc.at[didx.at[j]], add=True)

    load_and_fire(0, sidx_a, didx_a, rows_a, sem_a)

    def body(i, carry):
        load_and_fire(2 * i + 1, sidx_b, didx_b, rows_b, sem_b)
        drain_and_scatter(sidx_a, didx_a, rows_a, sem_a)
        load_and_fire(jnp.minimum(2 * i + 2, CHUNKS_PER_W - 1),
                      sidx_a, didx_a, rows_a, sem_a)
        drain_and_scatter(sidx_b, didx_b, rows_b, sem_b)
        return carry

    lax.fori_loop(0, CHUNKS_PER_W // 2, body, 0)
    for j in range(SUB):
        pltpu.make_async_copy(table.at[sidx_a.at[j]],
                              rows_a.at[pl.ds(j * 128, 128)], sem_a).wait()
    plsc.subcore_barrier()
    pltpu.sync_copy(acc.at[pl.ds(z0, ZROWS_PER_TILE)],
                    out.at[cid, pl.ds(z0, ZROWS_PER_TILE)])


def _deg_body(dst_r, zeros, ones, out, acc, didx, rows, sem):
    cid = lax.axis_index("c")
    sid = lax.axis_index("s")
    wid = cid * NS + sid
    z0 = sid * ZROWS_PER_TILE
    pltpu.sync_copy(zeros.at[pl.ds(z0, ZROWS_PER_TILE)],
                    acc.at[pl.ds(z0, ZROWS_PER_TILE)])
    pltpu.sync_copy(ones, rows)
    plsc.subcore_barrier()

    rbase = wid * ROWS_PER_W

    def chunk(k, carry):
        r0 = rbase + k * SUB
        pltpu.sync_copy(dst_r.at[pl.ds(r0, SUB)], didx)
        for j in range(SUB):
            pltpu.sync_copy(rows, acc.at[didx.at[j]], add=True)
        return carry

    lax.fori_loop(0, CHUNKS_PER_W, chunk, 0)
    plsc.subcore_barrier()
    pltpu.sync_copy(acc.at[pl.ds(z0, ZROWS_PER_TILE)],
                    out.at[cid, pl.ds(z0, ZROWS_PER_TILE)])


_PART = jax.ShapeDtypeStruct((NC, ACC_ROWS, LANES), jnp.float32)

_sc_params = pltpu.CompilerParams(use_tc_tiling_on_sc=False)

_agg = pl.kernel(
    _agg_body,
    out_type=_PART,
    mesh=_mesh,
    compiler_params=_sc_params,
    scratch_types=[
        pltpu.VMEM_SHARED((ACC_ROWS, LANES), jnp.float32),
        pltpu.VMEM((SUB, 128), jnp.int32),
        pltpu.VMEM((SUB, 128), jnp.int32),
        pltpu.VMEM((CHUNK_E, LANES), jnp.float32),
        pltpu.SemaphoreType.DMA,
        pltpu.VMEM((SUB, 128), jnp.int32),
        pltpu.VMEM((SUB, 128), jnp.int32),
        pltpu.VMEM((CHUNK_E, LANES), jnp.float32),
        pltpu.SemaphoreType.DMA,
    ],
)

_deg = pl.kernel(
    _deg_body,
    out_type=_PART,
    mesh=_mesh,
    compiler_params=_sc_params,
    scratch_types=[
        pltpu.VMEM_SHARED((ACC_ROWS, LANES), jnp.float32),
        pltpu.VMEM((SUB, 128), jnp.int32),
        pltpu.VMEM((128, LANES), jnp.float32),
        pltpu.SemaphoreType.DMA,
    ],
)

_BLK = 2000
_GRID = N_NODES // _BLK


def _tc1_body(deg_ref, x_ref, dinv_ref, y1_ref):
    d = deg_ref[0] + deg_ref[1] + 1.0
    dinv = lax.rsqrt(d)
    dinv_ref[...] = dinv
    y1_ref[...] = dinv * x_ref[...]


def _tc2_body(dinv_ref, agg_ref, y1_ref, w1_ref, b1_ref, y2lo_ref, y2hi_ref):
    dinv = dinv_ref[...]
    p1 = dinv * (agg_ref[0] + agg_ref[1] + y1_ref[...])
    h = jnp.dot(p1, w1_ref[...], preferred_element_type=jnp.float32)
    h = jnp.maximum(h + b1_ref[...], 0.0)
    y2lo_ref[...] = dinv * h[:, :LANES]
    y2hi_ref[...] = dinv * h[:, LANES:]


def _tc3_body(dinv_ref, alo_ref, ahi_ref, y2lo_ref, y2hi_ref, w2_ref,
              b2_ref, o_ref):
    dinv = dinv_ref[...]
    plo = dinv * (alo_ref[0] + alo_ref[1] + y2lo_ref[...])
    phi = dinv * (ahi_ref[0] + ahi_ref[1] + y2hi_ref[...])
    p2 = jnp.concatenate([plo, phi], axis=1)
    o = jnp.dot(p2, w2_ref[...], preferred_element_type=jnp.float32)
    o_ref[...] = jnp.maximum(o + b2_ref[...], 0.0)


def _vec_spec():
    return pl.BlockSpec((_BLK, LANES), lambda i: (i, 0))


def _part_spec():
    return pl.BlockSpec((NC, _BLK, LANES), lambda i: (0, i, 0))


def _full_spec(shape):
    return pl.BlockSpec(shape, lambda i: tuple(0 for _ in shape))


_tc1 = pl.pallas_call(
    _tc1_body,
    grid=(_GRID,),
    in_specs=[_part_spec(), _vec_spec()],
    out_specs=[_vec_spec(), _vec_spec()],
    out_shape=[jax.ShapeDtypeStruct((N_NODES, LANES), jnp.float32)] * 2,
)

_tc2 = pl.pallas_call(
    _tc2_body,
    grid=(_GRID,),
    in_specs=[_vec_spec(), _part_spec(), _vec_spec(),
              _full_spec((16, 32)), _full_spec((1, 32))],
    out_specs=[_vec_spec(), _vec_spec()],
    out_shape=[jax.ShapeDtypeStruct((N_NODES, LANES), jnp.float32)] * 2,
)

_tc3 = pl.pallas_call(
    _tc3_body,
    grid=(_GRID,),
    in_specs=[_vec_spec(), _part_spec(), _part_spec(), _vec_spec(),
              _vec_spec(), _full_spec((32, 64)), _full_spec((1, 64))],
    out_specs=pl.BlockSpec((_BLK, 64), lambda i: (i, 0)),
    out_shape=jax.ShapeDtypeStruct((N_NODES, 64), jnp.float32),
)


def kernel(x, edge_index, W1, b1, W2, b2):
    src = edge_index[0]
    dst = edge_index[1]
    pad = EPAD - N_EDGES
    src_p = jnp.concatenate([src, jnp.zeros((pad,), jnp.int32)])
    dst_p = jnp.concatenate([dst, jnp.full((pad,), N_NODES, jnp.int32)])
    src_r = src_p.reshape(ROWS, 128)
    dst_r = dst_p.reshape(ROWS, 128)
    zeros_sh = jnp.zeros((ACC_ROWS, LANES), jnp.float32)
    ones_blk = jnp.ones((128, LANES), jnp.float32)

    deg = _deg(dst_r, zeros_sh, ones_blk)
    dinv, y1 = _tc1(deg, x)
    agg1 = _agg(y1, src_r, dst_r, zeros_sh)
    y2lo, y2hi = _tc2(dinv, agg1, y1, W1, b1.reshape(1, 32))
    a2lo = _agg(y2lo, src_r, dst_r, zeros_sh)
    a2hi = _agg(y2hi, src_r, dst_r, zeros_sh)
    return _tc3(dinv, a2lo, a2hi, y2lo, y2hi, W2, b2.reshape(1, 64))


# speedup vs baseline: 54.7441x; 1.2594x over previous
import jax
import jax.numpy as jnp
from jax import lax
from jax.experimental import pallas as pl
from jax.experimental.pallas import tpu as pltpu
from jax.experimental.pallas import tpu_sc as plsc

N_NODES = 100000
N_EDGES = 3200000
LANES = 16

NC, NS = 2, 16
NW = NC * NS
SUB = 4
CHUNK_E = SUB * 128
CHUNKS_PER_W = 196
GCHUNK = 14
GROUPS = CHUNKS_PER_W // GCHUNK
GROWS = GCHUNK * SUB
DCHUNK = 28
DGROUPS = CHUNKS_PER_W // DCHUNK
DROWS = DCHUNK * SUB
EPAD = NW * CHUNKS_PER_W * CHUNK_E
ROWS = EPAD // 128
ROWS_PER_W = ROWS // NW
ACC_ROWS = 100096
ZROWS_PER_TILE = ACC_ROWS // NS

_mesh = plsc.VectorSubcoreMesh(
    core_axis_name="c", subcore_axis_name="s", num_cores=NC, num_subcores=NS
)


def _agg_body(table, edges, zeros, out, acc, idx,
              rows_a, sem_ga, sem_sa, rows_b, sem_gb, sem_sb):
    cid = lax.axis_index("c")
    sid = lax.axis_index("s")
    wid = cid * NS + sid
    z0 = sid * ZROWS_PER_TILE
    pltpu.sync_copy(zeros.at[pl.ds(z0, ZROWS_PER_TILE)],
                    acc.at[pl.ds(z0, ZROWS_PER_TILE)])
    plsc.subcore_barrier()

    rbase = wid * ROWS_PER_W

    def fire_g(k, rows, sem):
        for j in range(SUB):
            pltpu.async_copy(table.at[idx.at[k * SUB + j, 0]],
                             rows.at[pl.ds(j * 128, 128)], sem)

    def drain_g(rows, sem):
        for j in range(SUB):
            pltpu.make_async_copy(table.at[idx.at[0, 0]],
                                  rows.at[pl.ds(j * 128, 128)], sem).wait()

    def fire_s(k, rows, sem):
        for j in range(SUB):
            pltpu.async_copy(rows.at[pl.ds(j * 128, 128)],
                             acc.at[idx.at[k * SUB + j, 1]], sem, add=True)

    def drain_s(rows, sem):
        for j in range(SUB):
            pltpu.make_async_copy(rows.at[pl.ds(j * 128, 128)],
                                  acc.at[idx.at[0, 1]], sem).wait()

    def group(g, carry):
        pltpu.sync_copy(edges.at[pl.ds(rbase + g * GROWS, GROWS)], idx)
        fire_g(0, rows_a, sem_ga)
        fire_g(1, rows_b, sem_gb)

        def body(i, c2):
            a = 2 * i
            drain_g(rows_a, sem_ga)
            fire_s(a, rows_a, sem_sa)
            drain_g(rows_b, sem_gb)
            fire_s(a + 1, rows_b, sem_sb)
            drain_s(rows_a, sem_sa)

            @pl.when(i < GCHUNK // 2 - 1)
            def _():
                fire_g(a + 2, rows_a, sem_ga)

            drain_s(rows_b, sem_sb)

            @pl.when(i < GCHUNK // 2 - 1)
            def _():
                fire_g(a + 3, rows_b, sem_gb)

            return c2

        lax.fori_loop(0, GCHUNK // 2, body, 0)
        return carry

    lax.fori_loop(0, GROUPS, group, 0)
    plsc.subcore_barrier()
    pltpu.sync_copy(acc.at[pl.ds(z0, ZROWS_PER_TILE)],
                    out.at[cid, pl.ds(z0, ZROWS_PER_TILE)])


def _deg_body(dst_r, zeros, ones, out, acc, didx, rows, sem_s):
    cid = lax.axis_index("c")
    sid = lax.axis_index("s")
    wid = cid * NS + sid
    z0 = sid * ZROWS_PER_TILE
    pltpu.sync_copy(zeros.at[pl.ds(z0, ZROWS_PER_TILE)],
                    acc.at[pl.ds(z0, ZROWS_PER_TILE)])
    pltpu.sync_copy(ones, rows)
    plsc.subcore_barrier()

    rbase = wid * ROWS_PER_W

    def group(g, carry):
        pltpu.sync_copy(dst_r.at[pl.ds(rbase + g * DROWS, DROWS)], didx)

        def body(i, c2):
            for j in range(SUB):
                pltpu.async_copy(rows, acc.at[didx.at[i * SUB + j]],
                                 sem_s, add=True)

            @pl.when(i > 0)
            def _():
                for j in range(SUB):
                    pltpu.make_async_copy(rows, acc.at[didx.at[0]],
                                          sem_s).wait()

            return c2

        lax.fori_loop(0, DCHUNK, body, 0)
        for j in range(SUB):
            pltpu.make_async_copy(rows, acc.at[didx.at[0]], sem_s).wait()
        return carry

    lax.fori_loop(0, DGROUPS, group, 0)
    plsc.subcore_barrier()
    pltpu.sync_copy(acc.at[pl.ds(z0, ZROWS_PER_TILE)],
                    out.at[cid, pl.ds(z0, ZROWS_PER_TILE)])


_PART = jax.ShapeDtypeStruct((NC, ACC_ROWS, LANES), jnp.float32)

_sc_params = pltpu.CompilerParams(use_tc_tiling_on_sc=False)

_agg = pl.kernel(
    _agg_body,
    out_type=_PART,
    mesh=_mesh,
    compiler_params=_sc_params,
    scratch_types=[
        pltpu.VMEM_SHARED((ACC_ROWS, LANES), jnp.float32),
        pltpu.VMEM((GROWS, 2, 128), jnp.int32),
        pltpu.VMEM((CHUNK_E, LANES), jnp.float32),
        pltpu.SemaphoreType.DMA,
        pltpu.SemaphoreType.DMA,
        pltpu.VMEM((CHUNK_E, LANES), jnp.float32),
        pltpu.SemaphoreType.DMA,
        pltpu.SemaphoreType.DMA,
    ],
)

_deg = pl.kernel(
    _deg_body,
    out_type=_PART,
    mesh=_mesh,
    compiler_params=_sc_params,
    scratch_types=[
        pltpu.VMEM_SHARED((ACC_ROWS, LANES), jnp.float32),
        pltpu.VMEM((DROWS, 128), jnp.int32),
        pltpu.VMEM((128, LANES), jnp.float32),
        pltpu.SemaphoreType.DMA,
    ],
)

_BLK = 2000
_GRID = N_NODES // _BLK


def _tc1_body(deg_ref, x_ref, dinv_ref, y1_ref):
    d = deg_ref[0] + deg_ref[1] + 1.0
    dinv = lax.rsqrt(d)
    dinv_ref[...] = dinv
    y1_ref[...] = dinv * x_ref[...]


def _tc2_body(dinv_ref, agg_ref, y1_ref, w1_ref, b1_ref, y2lo_ref, y2hi_ref):
    dinv = dinv_ref[...]
    p1 = dinv * (agg_ref[0] + agg_ref[1] + y1_ref[...])
    h = jnp.dot(p1, w1_ref[...], preferred_element_type=jnp.float32)
    h = jnp.maximum(h + b1_ref[...], 0.0)
    y2lo_ref[...] = dinv * h[:, :LANES]
    y2hi_ref[...] = dinv * h[:, LANES:]


def _tc3_body(dinv_ref, alo_ref, ahi_ref, y2lo_ref, y2hi_ref, w2_ref,
              b2_ref, o_ref):
    dinv = dinv_ref[...]
    plo = dinv * (alo_ref[0] + alo_ref[1] + y2lo_ref[...])
    phi = dinv * (ahi_ref[0] + ahi_ref[1] + y2hi_ref[...])
    p2 = jnp.concatenate([plo, phi], axis=1)
    o = jnp.dot(p2, w2_ref[...], preferred_element_type=jnp.float32)
    o_ref[...] = jnp.maximum(o + b2_ref[...], 0.0)


def _vec_spec():
    return pl.BlockSpec((_BLK, LANES), lambda i: (i, 0))


def _part_spec():
    return pl.BlockSpec((NC, _BLK, LANES), lambda i: (0, i, 0))


def _full_spec(shape):
    return pl.BlockSpec(shape, lambda i: tuple(0 for _ in shape))


_tc1 = pl.pallas_call(
    _tc1_body,
    grid=(_GRID,),
    in_specs=[_part_spec(), _vec_spec()],
    out_specs=[_vec_spec(), _vec_spec()],
    out_shape=[jax.ShapeDtypeStruct((N_NODES, LANES), jnp.float32)] * 2,
)

_tc2 = pl.pallas_call(
    _tc2_body,
    grid=(_GRID,),
    in_specs=[_vec_spec(), _part_spec(), _vec_spec(),
              _full_spec((16, 32)), _full_spec((1, 32))],
    out_specs=[_vec_spec(), _vec_spec()],
    out_shape=[jax.ShapeDtypeStruct((N_NODES, LANES), jnp.float32)] * 2,
)

_tc3 = pl.pallas_call(
    _tc3_body,
    grid=(_GRID,),
    in_specs=[_vec_spec(), _part_spec(), _part_spec(), _vec_spec(),
              _vec_spec(), _full_spec((32, 64)), _full_spec((1, 64))],
    out_specs=pl.BlockSpec((_BLK, 64), lambda i: (i, 0)),
    out_shape=jax.ShapeDtypeStruct((N_NODES, 64), jnp.float32),
)


def kernel(x, edge_index, W1, b1, W2, b2):
    src = edge_index[0]
    dst = edge_index[1]
    pad = EPAD - N_EDGES
    src_p = jnp.concatenate([src, jnp.zeros((pad,), jnp.int32)])
    dst_p = jnp.concatenate([dst, jnp.full((pad,), N_NODES, jnp.int32)])
    src_r = src_p.reshape(ROWS, 128)
    dst_r = dst_p.reshape(ROWS, 128)
    edges = jnp.stack([src_r, dst_r], axis=1)
    zeros_sh = jnp.zeros((ACC_ROWS, LANES), jnp.float32)
    ones_blk = jnp.ones((128, LANES), jnp.float32)

    deg = _deg(dst_r, zeros_sh, ones_blk)
    dinv, y1 = _tc1(deg, x)
    agg1 = _agg(y1, edges, zeros_sh)
    y2lo, y2hi = _tc2(dinv, agg1, y1, W1, b1.reshape(1, 32))
    a2lo = _agg(y2lo, edges, zeros_sh)
    a2hi = _agg(y2hi, edges, zeros_sh)
    return _tc3(dinv, a2lo, a2hi, y2lo, y2hi, W2, b2.reshape(1, 64))
